# Initial kernel scaffold; baseline (speedup 1.0000x reference)
import jax
import jax.numpy as jnp
from jax.experimental import pallas as pl


def kernel(xyz0, idx0, idx5, idx7, idx9, idx11, W_all, b_all, W_m4, b_m4, W_m3, b_m3, W_m2, b_m2, W_m1, b_m1, W_m0, b_m0, W_p4, b_p4, W_p3, b_p3, W_p2, b_p2, W_p1, b_p1, W_p0, b_p0):
    raise NotImplementedError("write your pallas kernel here")


# speedup vs baseline: 2.2510x; 2.2510x over previous
import functools

import jax
import jax.numpy as jnp
from jax import lax
from jax.experimental import pallas as pl
from jax.experimental.pallas import tpu as pltpu
from jax.experimental.pallas import tpu_sc as plsc

N0, N1, N2, N3, N4 = 32768, 8192, 2048, 512, 128
C = 256
XP = 16
NC, NS = 2, 16
NW = NC * NS

_SC_MESH = dict(mesh=plsc.VectorSubcoreMesh(core_axis_name="c", subcore_axis_name="s"))


def _wid():
    return lax.axis_index("c") * NS + lax.axis_index("s")


GP = 128


@functools.partial(
    pl.kernel,
    out_type=(
        jax.ShapeDtypeStruct((N1, GP), jnp.float32),
        jax.ShapeDtypeStruct((N2, GP), jnp.float32),
        jax.ShapeDtypeStruct((N3, GP), jnp.float32),
        jax.ShapeDtypeStruct((N4, GP), jnp.float32),
    ),
    scratch_types=[
        pltpu.VMEM((N1 // NW,), jnp.int32),
        pltpu.VMEM((N1 // NW, GP), jnp.float32),
        pltpu.SemaphoreType.DMA,
    ],
    **_SC_MESH,
)
def _sc_gather_xyz(x0p, i11, i9, i7, i5, o1, o2, o3, o4, idx_v, rows_v, sem):
    w = _wid()

    def gather(idx_hbm, out_hbm, n):
        per = n // NW
        base = w * per
        pltpu.sync_copy(idx_hbm.at[pl.ds(base, per)], idx_v.at[pl.ds(0, per)])
        pltpu.async_copy(
            x0p.at[idx_v.at[pl.ds(0, per)]], rows_v.at[pl.ds(0, per)], sem
        ).wait()
        pltpu.sync_copy(rows_v.at[pl.ds(0, per)], out_hbm.at[pl.ds(base, per)])

    gather(i11, o1, N1)
    gather(i9, o2, N2)
    gather(i7, o3, N3)
    @pl.when(w < 16)
    def _():
        per = N4 // 16
        base = w * per
        pltpu.sync_copy(i5.at[pl.ds(base, per)], idx_v.at[pl.ds(0, per)])
        pltpu.async_copy(
            x0p.at[idx_v.at[pl.ds(0, per)]], rows_v.at[pl.ds(0, per)], sem
        ).wait()
        pltpu.sync_copy(rows_v.at[pl.ds(0, per)], o4.at[pl.ds(base, per)])


def _argmin_rows(q, rT):
    dot = jnp.dot(q, rT, preferred_element_type=jnp.float32)
    qn = jnp.sum(q * q, axis=1, keepdims=True)
    rn = jnp.sum(rT * rT, axis=0, keepdims=True)
    d = qn + rn - 2.0 * dot
    minv = jnp.min(d, axis=1, keepdims=True)
    ii = lax.broadcasted_iota(jnp.int32, d.shape, 1)
    return jnp.min(jnp.where(d <= minv, ii, jnp.int32(2**30)), axis=1)


_KNN_BLK = 512


def _tc_knn12_body(q_ref, rT_ref, o_ref):
    o_ref[0, 0, :] = _argmin_rows(q_ref[...], rT_ref[...])


def _tc_knn12(xyz1p, xyz2pT):
    nblk = N1 // _KNN_BLK
    out = pl.pallas_call(
        _tc_knn12_body,
        grid=(nblk,),
        in_specs=[
            pl.BlockSpec((_KNN_BLK, XP), lambda i: (i, 0)),
            pl.BlockSpec((XP, N2), lambda i: (0, 0)),
        ],
        out_specs=pl.BlockSpec((1, 1, _KNN_BLK), lambda i: (i, 0, 0)),
        out_shape=jax.ShapeDtypeStruct((nblk, 1, _KNN_BLK), jnp.int32),
    )(xyz1p, xyz2pT)
    return out.reshape(N1)


def _tc_levels_body(
    x0p8k_ref, x1p_ref, x2p_ref, x3p_ref, x4p_ref, x3T_ref, x4T_ref,
    wallp_ref, wm4p_ref, wm3p_ref, wm2p_ref, wm1p_ref,
    wp4_ref, wp3_ref, wp2_ref, wp1_ref, bias_ref,
    f4o_ref, f3o_ref, f2o_ref, g2o_ref, q1o_ref,
):
    x0p = x0p8k_ref[...]
    x4p, x3p, x2p, x1p = x4p_ref[...], x3p_ref[...], x2p_ref[...], x1p_ref[...]
    wallp = wallp_ref[...]
    b_all = bias_ref[0:1, :]

    def dot(a, b):
        return jnp.dot(a, b, preferred_element_type=jnp.float32)

    wp4a, wp4b = wp4_ref[:C, :], wp4_ref[C:, :]
    f4 = dot(x0p[:N4], wallp) + b_all
    m4 = jnp.max(f4, axis=0, keepdims=True)
    M4 = dot(wm4p_ref[...], wp4a)
    A4 = dot(wallp, wp4b)
    c4 = dot(bias_ref[1:2, :], wp4a) + dot(b_all, wp4b) + bias_ref[5:6, :]
    feat4 = dot(m4, wp4a) + dot(x4p, M4) + dot(x0p[:N4], A4) + c4
    f4o_ref[...] = feat4

    wp3a, wp3b = wp3_ref[:C, :], wp3_ref[C:, :]
    M3 = dot(wm3p_ref[...], wp3a)
    A3 = dot(wallp, wp3b)
    c3 = dot(bias_ref[2:3, :], wp3a) + dot(b_all, wp3b) + bias_ref[6:7, :]
    G4 = dot(feat4, wp3a) - dot(x4p, M3) + c3
    k34 = _argmin_rows(x3p, x4T_ref[...])
    oh34 = (k34[:, None] == lax.broadcasted_iota(jnp.int32, (N3, N4), 1)).astype(
        jnp.float32
    )
    feat3 = dot(oh34, G4) + dot(x3p, M3) + dot(x0p[:N3], A3)
    f3o_ref[...] = feat3

    wp2a, wp2b = wp2_ref[:C, :], wp2_ref[C:, :]
    M2 = dot(wm2p_ref[...], wp2a)
    A2 = dot(wallp, wp2b)
    c2 = dot(bias_ref[3:4, :], wp2a) + dot(b_all, wp2b) + bias_ref[7:8, :]
    G3 = dot(feat3, wp2a) - dot(x3p, M2) + c2
    k23 = _argmin_rows(x2p, x3T_ref[...])
    oh23 = (k23[:, None] == lax.broadcasted_iota(jnp.int32, (N2, N3), 1)).astype(
        jnp.float32
    )
    feat2 = dot(oh23, G3) + dot(x2p, M2) + dot(x0p[:N2], A2)
    f2o_ref[...] = feat2

    wp1a, wp1b = wp1_ref[:C, :], wp1_ref[C:, :]
    M1 = dot(wm1p_ref[...], wp1a)
    A1 = dot(wallp, wp1b)
    c1 = dot(bias_ref[4:5, :], wp1a) + dot(b_all, wp1b) + bias_ref[8:9, :]
    g2o_ref[...] = dot(feat2, wp1a) - dot(x2p, M1) + c1
    q1o_ref[...] = dot(x1p, M1) + dot(x0p, A1)


def _tc_levels(x0p8k, x1p, x2p, x3p, x4p, x3T, x4T, wallp, wm4p, wm3p, wm2p,
               wm1p, wp4, wp3, wp2, wp1, bias):
    return pl.pallas_call(
        _tc_levels_body,
        out_shape=(
            jax.ShapeDtypeStruct((N4, C), jnp.float32),
            jax.ShapeDtypeStruct((N3, C), jnp.float32),
            jax.ShapeDtypeStruct((N2, C), jnp.float32),
            jax.ShapeDtypeStruct((N2, C), jnp.float32),
            jax.ShapeDtypeStruct((N1, C), jnp.float32),
        ),
    )(x0p8k, x1p, x2p, x3p, x4p, x3T, x4T, wallp, wm4p, wm3p, wm2p, wm1p,
      wp4, wp3, wp2, wp1, bias)


_F1_SUB = 128


@functools.partial(
    pl.kernel,
    out_type=jax.ShapeDtypeStruct((N1, C), jnp.float32),
    scratch_types=[
        pltpu.VMEM((_F1_SUB,), jnp.int32),
        pltpu.VMEM((_F1_SUB, C), jnp.float32),
        pltpu.VMEM((_F1_SUB, C), jnp.float32),
        pltpu.SemaphoreType.DMA,
    ],
    **_SC_MESH,
)
def _sc_feat1(g2, k12, q1, out, idx_v, g_v, q_v, sem):
    w = _wid()
    per = N1 // NW

    def sub(s, _):
        base = w * per + s * _F1_SUB
        pltpu.sync_copy(k12.at[pl.ds(base, _F1_SUB)], idx_v)
        cp = pltpu.async_copy(g2.at[idx_v], g_v, sem)
        pltpu.sync_copy(q1.at[pl.ds(base, _F1_SUB)], q_v)
        cp.wait()

        def row(r, _):
            for c in range(C // 16):
                sl = pl.ds(c * 16, 16)
                g_v[r, sl] = g_v[r, sl] + q_v[r, sl]
            return 0

        lax.fori_loop(0, _F1_SUB, row, 0)
        pltpu.sync_copy(g_v, out.at[pl.ds(base, _F1_SUB)])
        return 0

    lax.fori_loop(0, per // _F1_SUB, sub, 0)


def _tc_g1_body(f1_ref, x1p_ref, wallp_ref, wm0p_ref, wp0_ref, bias0_ref,
                g1o_ref, s0o_ref):
    def dot(a, b):
        return jnp.dot(a, b, preferred_element_type=jnp.float32)

    wp0a, wp0b = wp0_ref[:C, :], wp0_ref[C:, :]
    M0 = dot(wm0p_ref[...], wp0a)
    A0 = dot(wallp_ref[...], wp0b)
    c0 = (dot(bias0_ref[1:2, :], wp0a) + dot(bias0_ref[0:1, :], wp0b)
          + bias0_ref[2:3, :])
    g1o_ref[...] = dot(f1_ref[...], wp0a) - dot(x1p_ref[...], M0) + c0
    s0o_ref[...] = M0 + A0


def _tc_g1(feat1, x1p, wallp, wm0p, wp0, bias0):
    return pl.pallas_call(
        _tc_g1_body,
        out_shape=(
            jax.ShapeDtypeStruct((N1, C), jnp.float32),
            jax.ShapeDtypeStruct((XP, C), jnp.float32),
        ),
    )(feat1, x1p, wallp, wm0p, wp0, bias0)


_F0_SUB = 128


@functools.partial(
    pl.kernel,
    out_type=jax.ShapeDtypeStruct((N0, C), jnp.float32),
    scratch_types=[
        pltpu.VMEM((_F0_SUB,), jnp.int32),
        pltpu.VMEM((_F0_SUB, C), jnp.float32),
        pltpu.VMEM((_F0_SUB, XP), jnp.float32),
        pltpu.VMEM((3 * C,), jnp.float32),
        pltpu.SemaphoreType.DMA,
    ],
    **_SC_MESH,
)
def _sc_feat0(g1, idx0, x0p, s0flat, out, idx_v, g_v, x_v, s_v, sem):
    w = _wid()
    per = N0 // NW
    pltpu.sync_copy(s0flat, s_v)
    s0 = [s_v[pl.ds(c * 16, 16)] for c in range(C // 16)]
    s1 = [s_v[pl.ds(C + c * 16, 16)] for c in range(C // 16)]
    s2 = [s_v[pl.ds(2 * C + c * 16, 16)] for c in range(C // 16)]

    def sub(s, _):
        base = w * per + s * _F0_SUB
        pltpu.sync_copy(idx0.at[pl.ds(base, _F0_SUB)], idx_v)
        cp = pltpu.async_copy(g1.at[idx_v], g_v, sem)
        pltpu.sync_copy(x0p.at[pl.ds(base, _F0_SUB)], x_v)
        cp.wait()

        def row(r, _):
            xrow = x_v[r, pl.ds(0, 16)]
            x, y, z = xrow[0], xrow[1], xrow[2]
            for c in range(C // 16):
                sl = pl.ds(c * 16, 16)
                g_v[r, sl] = g_v[r, sl] + x * s0[c] + y * s1[c] + z * s2[c]
            return 0

        lax.fori_loop(0, _F0_SUB, row, 0)
        pltpu.sync_copy(g_v, out.at[pl.ds(base, _F0_SUB)])
        return 0

    lax.fori_loop(0, per // _F0_SUB, sub, 0)


def kernel(xyz0, idx0, idx5, idx7, idx9, idx11, W_all, b_all, W_m4, b_m4,
           W_m3, b_m3, W_m2, b_m2, W_m1, b_m1, W_m0, b_m0, W_p4, b_p4,
           W_p3, b_p3, W_p2, b_p2, W_p1, b_p1, W_p0, b_p0):
    f32 = jnp.float32
    x0p = jnp.pad(xyz0.astype(f32), ((0, 0), (0, XP - 3)))

    def padw(w):
        return jnp.pad(w.astype(f32), ((0, XP - 3), (0, 0)))

    wallp = padw(W_all)
    i0 = idx0.astype(jnp.int32)
    i5, i7, i9, i11 = (i.astype(jnp.int32) for i in (idx5, idx7, idx9, idx11))

    x0g = jnp.pad(xyz0.astype(f32), ((0, 0), (0, GP - 3)))
    x1g, x2g, x3g, x4g = _sc_gather_xyz(x0g, i11, i9, i7, i5)
    x1p, x2p, x3p, x4p = (g[:, :XP] for g in (x1g, x2g, x3g, x4g))

    k12 = _tc_knn12(x1p, x2p.T)

    bias = jnp.stack([b_all, b_m4, b_m3, b_m2, b_m1, b_p4, b_p3, b_p2, b_p1])
    feat4, feat3, feat2, G2, Q1 = _tc_levels(
        x0p[:N1], x1p, x2p, x3p, x4p, x3p.T, x4p.T,
        wallp, padw(W_m4), padw(W_m3), padw(W_m2), padw(W_m1),
        W_p4, W_p3, W_p2, W_p1, bias)

    feat1 = _sc_feat1(G2, k12, Q1)

    bias0 = jnp.stack([b_all, b_m0, b_p0])
    G1, S0 = _tc_g1(feat1, x1p, wallp, padw(W_m0), W_p0, bias0)

    feat0 = _sc_feat0(G1, i0, x0p, S0[:3].reshape(3 * C))

    return (feat4, feat3, feat2, feat1, feat0)
